# Initial kernel scaffold; baseline (speedup 1.0000x reference)
import jax
import jax.numpy as jnp
from jax.experimental import pallas as pl


def kernel(x, edge_index_rel0, edge_index_rel1, W_rel0, W_rel1, W_loop, b_loop):
    raise NotImplementedError("write your pallas kernel here")


# speedup vs baseline: 2.6859x; 2.6859x over previous
import functools

import jax
import jax.numpy as jnp
from jax import lax
from jax.experimental import pallas as pl
from jax.experimental.pallas import tpu as pltpu
from jax.experimental.pallas import tpu_sc as plsc

N_NODES = 10000
D = 128

NC = 2
NS = 16
CHUNK = 128

N_PAD = 10240
ROWS_PER_TILE = N_PAD // NS
CROWS = N_PAD // D


def _sc_aggregate(x, eye, src_all, dst_all, dhi_all, dlo_all, zacc, n_chunks):
    mesh = plsc.VectorSubcoreMesh(
        core_axis_name="c", subcore_axis_name="s", num_cores=NC, num_subcores=NS
    )

    @functools.partial(
        pl.kernel,
        out_type=[
            jax.ShapeDtypeStruct((NC, N_PAD, D), jnp.float32),
            jax.ShapeDtypeStruct((NC, CROWS, D), jnp.float32),
        ],
        mesh=mesh,
        scratch_types=[
            pltpu.VMEM((CHUNK,), jnp.int32),
            pltpu.VMEM((CHUNK,), jnp.int32),
            pltpu.VMEM((CHUNK,), jnp.int32),
            pltpu.VMEM((CHUNK,), jnp.int32),
            pltpu.VMEM((CHUNK, D), jnp.float32),
            pltpu.VMEM_SHARED((N_PAD, D), jnp.float32),
            pltpu.VMEM_SHARED((CROWS, D), jnp.float32),
            pltpu.SemaphoreType.DMA,
        ],
    )
    def agg(x_hbm, eye_hbm, src_hbm, dst_hbm, dhi_hbm, dlo_hbm, zacc_hbm,
            sums_hbm, cnts_hbm,
            src_v, dst_v, dhi_v, dlo_v, msg_v, acc_sh, cnt_sh, sem):
        c = lax.axis_index("c")
        s = lax.axis_index("s")
        row0 = s * ROWS_PER_TILE
        n_sub = ROWS_PER_TILE // CHUNK

        pltpu.sync_copy(zacc_hbm, msg_v)
        for k in range(n_sub):
            pltpu.sync_copy(msg_v, acc_sh.at[pl.ds(row0 + k * CHUNK, CHUNK)])

        @pl.when(s == 0)
        def _():
            pltpu.sync_copy(msg_v.at[pl.ds(0, CROWS)], cnt_sh)

        plsc.subcore_barrier()

        def body(j, carry):
            pltpu.sync_copy(src_hbm.at[c, s, j], src_v)
            pltpu.sync_copy(dst_hbm.at[c, s, j], dst_v)
            pltpu.sync_copy(dhi_hbm.at[c, s, j], dhi_v)
            pltpu.sync_copy(dlo_hbm.at[c, s, j], dlo_v)
            pltpu.async_copy(x_hbm.at[src_v], msg_v, sem).wait()
            pltpu.sync_copy(msg_v, acc_sh.at[dst_v], add=True)
            pltpu.async_copy(eye_hbm.at[dlo_v], msg_v, sem).wait()
            pltpu.sync_copy(msg_v, cnt_sh.at[dhi_v], add=True)
            return carry

        lax.fori_loop(0, n_chunks, body, 0)
        plsc.subcore_barrier()

        for k in range(n_sub):
            sl = pl.ds(row0 + k * CHUNK, CHUNK)
            pltpu.sync_copy(acc_sh.at[sl], msg_v)
            pltpu.sync_copy(msg_v, sums_hbm.at[c, sl])

        @pl.when(s == 0)
        def _():
            pltpu.sync_copy(cnt_sh, msg_v.at[pl.ds(0, CROWS)])
            pltpu.sync_copy(msg_v.at[pl.ds(0, CROWS)], cnts_hbm.at[c])

    return agg(x, eye, src_all, dst_all, dhi_all, dlo_all, zacc)


def _tc_finish_body(s0_ref, s1_ref, c0_ref, c1_ref, x_ref, w0_ref, w1_ref,
                    wl_ref, b_ref, out_ref):
    inv0 = 1.0 / jnp.maximum(c0_ref[...], 1.0)
    inv1 = 1.0 / jnp.maximum(c1_ref[...], 1.0)
    m0 = s0_ref[0] * inv0
    m1 = s1_ref[0] * inv1
    acc = jnp.dot(m0, w0_ref[...], preferred_element_type=jnp.float32)
    acc += jnp.dot(m1, w1_ref[...], preferred_element_type=jnp.float32)
    acc += jnp.dot(x_ref[...], wl_ref[...], preferred_element_type=jnp.float32)
    acc += b_ref[...]
    out_ref[...] = jnp.maximum(acc, 0.0)


def _tc_finish(sums, cnt0, cnt1, x, W_rel0, W_rel1, W_loop, b_loop):
    B = 1000
    grid = (N_NODES // B,)
    return pl.pallas_call(
        _tc_finish_body,
        grid=grid,
        in_specs=[
            pl.BlockSpec((1, B, D), lambda i: (0, i, 0)),
            pl.BlockSpec((1, B, D), lambda i: (1, i, 0)),
            pl.BlockSpec((B, 1), lambda i: (i, 0)),
            pl.BlockSpec((B, 1), lambda i: (i, 0)),
            pl.BlockSpec((B, D), lambda i: (i, 0)),
            pl.BlockSpec((D, D), lambda i: (0, 0)),
            pl.BlockSpec((D, D), lambda i: (0, 0)),
            pl.BlockSpec((D, D), lambda i: (0, 0)),
            pl.BlockSpec((1, D), lambda i: (0, 0)),
        ],
        out_specs=pl.BlockSpec((B, D), lambda i: (i, 0)),
        out_shape=jax.ShapeDtypeStruct((N_NODES, D), jnp.float32),
    )(sums, sums, cnt0, cnt1, x, W_rel0, W_rel1, W_loop, b_loop.reshape(1, D))


def kernel(x, edge_index_rel0, edge_index_rel1, W_rel0, W_rel1, W_loop, b_loop):
    n_edges = edge_index_rel0.shape[1]
    per_tile = -(-n_edges // (NS * CHUNK)) * CHUNK
    n_chunks = per_tile // CHUNK
    e_pad = per_tile * NS
    pad = e_pad - n_edges

    def prep(ei):
        src = ei[0].astype(jnp.int32)
        dst = ei[1].astype(jnp.int32)
        src = jnp.concatenate([src, jnp.zeros((pad,), jnp.int32)])
        dst = jnp.concatenate([dst, jnp.full((pad,), N_NODES, jnp.int32)])
        sh = (NS, n_chunks, CHUNK)
        return (src.reshape(sh), dst.reshape(sh),
                (dst >> 7).reshape(sh), (dst & 127).reshape(sh))

    s0, d0, h0, l0 = prep(edge_index_rel0)
    s1, d1, h1, l1 = prep(edge_index_rel1)
    src_all = jnp.stack([s0, s1])
    dst_all = jnp.stack([d0, d1])
    dhi_all = jnp.stack([h0, h1])
    dlo_all = jnp.stack([l0, l1])

    zacc = jnp.zeros((CHUNK, D), jnp.float32)
    eye = jnp.eye(D, dtype=jnp.float32)

    sums, cnts = _sc_aggregate(x.astype(jnp.float32), eye, src_all, dst_all,
                               dhi_all, dlo_all, zacc, n_chunks)
    cnt_flat = cnts.reshape(NC, N_PAD)[:, :N_NODES]
    cnt0 = cnt_flat[0].reshape(N_NODES, 1)
    cnt1 = cnt_flat[1].reshape(N_NODES, 1)
    return _tc_finish(sums, cnt0, cnt1, x, W_rel0, W_rel1, W_loop, b_loop)


# speedup vs baseline: 3.1925x; 1.1886x over previous
import functools

import jax
import jax.numpy as jnp
from jax import lax
from jax.experimental import pallas as pl
from jax.experimental.pallas import tpu as pltpu
from jax.experimental.pallas import tpu_sc as plsc

N_NODES = 10000
D = 128

NC = 2
NS = 16
CHUNK = 104

N_PAD = 10112
ROWS_PER_TILE = N_PAD // NS
CROWS = 80


def _sc_aggregate(x, eye, idx_all, zacc, n_chunks):
    mesh = plsc.VectorSubcoreMesh(
        core_axis_name="c", subcore_axis_name="s", num_cores=NC, num_subcores=NS
    )

    @functools.partial(
        pl.kernel,
        out_type=[
            jax.ShapeDtypeStruct((NC, N_PAD, D), jnp.float32),
            jax.ShapeDtypeStruct((NC, CROWS, D), jnp.float32),
        ],
        mesh=mesh,
        scratch_types=[
            pltpu.VMEM((4, CHUNK), jnp.int32),
            pltpu.VMEM((4, CHUNK), jnp.int32),
            pltpu.VMEM((CHUNK, D), jnp.float32),
            pltpu.VMEM((CHUNK, D), jnp.float32),
            pltpu.VMEM_SHARED((N_PAD, D), jnp.float32),
            pltpu.VMEM_SHARED((CROWS, D), jnp.float32),
            pltpu.VMEM_SHARED((D, D), jnp.float32),
            pltpu.SemaphoreType.DMA,
            pltpu.SemaphoreType.DMA,
        ],
    )
    def agg(x_hbm, eye_hbm, idx_hbm, zacc_hbm,
            sums_hbm, cnts_hbm,
            ida, idb, bufa, bufb, acc_sh, cnt_sh, eye_sh, sema, semb):
        c = lax.axis_index("c")
        s = lax.axis_index("s")
        row0 = s * ROWS_PER_TILE

        pltpu.sync_copy(zacc_hbm, bufa)
        for k in range(6):
            pltpu.sync_copy(bufa, acc_sh.at[pl.ds(row0 + k * CHUNK, CHUNK)])
        pltpu.sync_copy(bufa.at[pl.ds(0, 8)],
                        acc_sh.at[pl.ds(row0 + 6 * CHUNK, 8)])

        @pl.when(s == 0)
        def _():
            pltpu.sync_copy(bufa.at[pl.ds(0, CROWS)], cnt_sh)
            pltpu.sync_copy(eye_hbm.at[pl.ds(0, CHUNK)], bufb)
            pltpu.sync_copy(bufb, eye_sh.at[pl.ds(0, CHUNK)])
            pltpu.sync_copy(eye_hbm.at[pl.ds(CHUNK, D - CHUNK)],
                            bufb.at[pl.ds(0, D - CHUNK)])
            pltpu.sync_copy(bufb.at[pl.ds(0, D - CHUNK)],
                            eye_sh.at[pl.ds(CHUNK, D - CHUNK)])

        plsc.subcore_barrier()

        def pair(k, carry):
            j0 = 2 * k
            j1 = j0 + 1
            pltpu.sync_copy(idx_hbm.at[c, s, j0], ida)
            cpa = pltpu.async_copy(x_hbm.at[ida.at[0]], bufa, sema)
            pltpu.sync_copy(idx_hbm.at[c, s, j1], idb)
            cpb = pltpu.async_copy(x_hbm.at[idb.at[0]], bufb, semb)
            cpa.wait()
            pltpu.sync_copy(bufa, acc_sh.at[ida.at[1]], add=True)
            pltpu.sync_copy(eye_sh.at[ida.at[2]], bufa)
            pltpu.sync_copy(bufa, cnt_sh.at[ida.at[3]], add=True)
            cpb.wait()
            pltpu.sync_copy(bufb, acc_sh.at[idb.at[1]], add=True)
            pltpu.sync_copy(eye_sh.at[idb.at[2]], bufb)
            pltpu.sync_copy(bufb, cnt_sh.at[idb.at[3]], add=True)
            return carry

        lax.fori_loop(0, n_chunks // 2, pair, 0)
        plsc.subcore_barrier()

        for k in range(6):
            sl = pl.ds(row0 + k * CHUNK, CHUNK)
            pltpu.sync_copy(acc_sh.at[sl], bufa)
            pltpu.sync_copy(bufa, sums_hbm.at[c, sl])
        sl = pl.ds(row0 + 6 * CHUNK, 8)
        pltpu.sync_copy(acc_sh.at[sl], bufa.at[pl.ds(0, 8)])
        pltpu.sync_copy(bufa.at[pl.ds(0, 8)], sums_hbm.at[c, sl])

        @pl.when(s == 0)
        def _():
            pltpu.sync_copy(cnt_sh, bufb.at[pl.ds(0, CROWS)])
            pltpu.sync_copy(bufb.at[pl.ds(0, CROWS)], cnts_hbm.at[c])

    return agg(x, eye, idx_all, zacc)


def _tc_finish_body(s0_ref, s1_ref, c0_ref, c1_ref, x_ref, w0_ref, w1_ref,
                    wl_ref, b_ref, out_ref):
    inv0 = 1.0 / jnp.maximum(c0_ref[...], 1.0)
    inv1 = 1.0 / jnp.maximum(c1_ref[...], 1.0)
    m0 = s0_ref[0] * inv0
    m1 = s1_ref[0] * inv1
    acc = jnp.dot(m0, w0_ref[...], preferred_element_type=jnp.float32)
    acc += jnp.dot(m1, w1_ref[...], preferred_element_type=jnp.float32)
    acc += jnp.dot(x_ref[...], wl_ref[...], preferred_element_type=jnp.float32)
    acc += b_ref[...]
    out_ref[...] = jnp.maximum(acc, 0.0)


def _tc_finish(sums, cnt0, cnt1, x, W_rel0, W_rel1, W_loop, b_loop):
    B = 1000
    grid = (N_NODES // B,)
    return pl.pallas_call(
        _tc_finish_body,
        grid=grid,
        in_specs=[
            pl.BlockSpec((1, B, D), lambda i: (0, i, 0)),
            pl.BlockSpec((1, B, D), lambda i: (1, i, 0)),
            pl.BlockSpec((B, 1), lambda i: (i, 0)),
            pl.BlockSpec((B, 1), lambda i: (i, 0)),
            pl.BlockSpec((B, D), lambda i: (i, 0)),
            pl.BlockSpec((D, D), lambda i: (0, 0)),
            pl.BlockSpec((D, D), lambda i: (0, 0)),
            pl.BlockSpec((D, D), lambda i: (0, 0)),
            pl.BlockSpec((1, D), lambda i: (0, 0)),
        ],
        out_specs=pl.BlockSpec((B, D), lambda i: (i, 0)),
        out_shape=jax.ShapeDtypeStruct((N_NODES, D), jnp.float32),
    )(sums, sums, cnt0, cnt1, x, W_rel0, W_rel1, W_loop, b_loop.reshape(1, D))


def kernel(x, edge_index_rel0, edge_index_rel1, W_rel0, W_rel1, W_loop, b_loop):
    n_edges = edge_index_rel0.shape[1]
    per_tile = -(-n_edges // (NS * 2 * CHUNK)) * 2 * CHUNK
    n_chunks = per_tile // CHUNK
    e_pad = per_tile * NS
    pad = e_pad - n_edges

    def prep(ei):
        src = ei[0].astype(jnp.int32)
        dst = ei[1].astype(jnp.int32)
        src = jnp.concatenate([src, jnp.zeros((pad,), jnp.int32)])
        dst = jnp.concatenate([dst, jnp.full((pad,), N_NODES, jnp.int32)])
        sh = (NS, n_chunks, CHUNK)
        return jnp.stack([src.reshape(sh), dst.reshape(sh),
                          (dst & 127).reshape(sh), (dst >> 7).reshape(sh)],
                         axis=2)

    idx_all = jnp.stack([prep(edge_index_rel0), prep(edge_index_rel1)])

    zacc = jnp.zeros((CHUNK, D), jnp.float32)
    eye = jnp.eye(D, dtype=jnp.float32)

    sums, cnts = _sc_aggregate(x.astype(jnp.float32), eye, idx_all, zacc,
                               n_chunks)
    cnt_flat = cnts.reshape(NC, CROWS * D)[:, :N_NODES]
    cnt0 = cnt_flat[0].reshape(N_NODES, 1)
    cnt1 = cnt_flat[1].reshape(N_NODES, 1)
    return _tc_finish(sums, cnt0, cnt1, x, W_rel0, W_rel1, W_loop, b_loop)
